# Initial kernel scaffold; baseline (speedup 1.0000x reference)
import jax
import jax.numpy as jnp
from jax.experimental import pallas as pl


def kernel(chosen_types, song_tokens, seq_scores):
    raise NotImplementedError("write your pallas kernel here")


# speedup vs baseline: 845.1284x; 845.1284x over previous
import jax
import jax.numpy as jnp
from jax.experimental import pallas as pl
from jax.experimental.pallas import tpu as pltpu

B = 8
S1 = 2047
TOTAL = 1391
INST_START = 852
WIN_LO = 768
WIN_HI = 1024
WOFF = INST_START - WIN_LO
TS = 256
NSB = 8
TOK_PAD = 2304
DUMMY = 300

NEG = -1e9


def _apply_body(tok_ref, ctq_ref, x_ref, o_ref, carry_ref):
    s = pl.program_id(1)

    toks = tok_ref[0, pl.ds(s * TS + 1, TS), :]
    lanes = jax.lax.broadcasted_iota(jnp.int32, (TS, 256), 1)
    pen = jnp.where(lanes == toks + WOFF, 0.0, 1.0)

    rows = jax.lax.broadcasted_iota(jnp.int32, (TS, 256), 0)
    m = pen
    sh = 1
    while sh < TS:
        rolled = pltpu.roll(m, sh, axis=0)
        m = jnp.minimum(m, jnp.where(rows < sh, 1.0, rolled))
        sh *= 2

    @pl.when(s == 0)
    def _():
        tok0 = tok_ref[0, 0, 0]
        init = jnp.where(
            jax.lax.broadcasted_iota(jnp.int32, (8, 256), 1) == tok0 + WOFF,
            0.0, 1.0)
        carry_ref[...] = init

    m = jnp.minimum(m, carry_ref[0:1, :])
    carry_ref[...] = jnp.broadcast_to(m[TS - 1:TS, :], (8, 256))

    keep = jnp.maximum(m, ctq_ref[0])

    o_ref[0, :, :WIN_LO] = x_ref[0, :, :WIN_LO]
    o_ref[0, :, WIN_LO:WIN_HI] = jnp.where(
        keep > 0.5, x_ref[0, :, WIN_LO:WIN_HI], NEG)
    o_ref[0, :, WIN_HI:] = x_ref[0, :, WIN_HI:]


def _build(interpret=False):
    return pl.pallas_call(
        _apply_body,
        grid=(B, NSB),
        in_specs=[
            pl.BlockSpec((1, TOK_PAD, 1), lambda b, s: (b, 0, 0)),
            pl.BlockSpec((1, TS, 1), lambda b, s: (b, s, 0)),
            pl.BlockSpec((1, TS, TOTAL), lambda b, s: (b, s, 0)),
        ],
        out_specs=pl.BlockSpec((1, TS, TOTAL), lambda b, s: (b, s, 0)),
        out_shape=jax.ShapeDtypeStruct((B, S1, TOTAL), jnp.float32),
        scratch_shapes=[pltpu.VMEM((8, 256), jnp.float32)],
        interpret=interpret,
    )


def kernel(chosen_types, song_tokens, seq_scores):
    song = song_tokens.astype(jnp.int32)
    tok = jnp.where(song[:, :, 0] == 1, song[:, :, 6], DUMMY)
    tok = jnp.pad(tok, ((0, 0), (0, TOK_PAD - S1)), constant_values=DUMMY)
    tok = tok[..., None]
    ctq = jnp.where(chosen_types.astype(jnp.int32) == 1, 0.0, 1.0)
    ctq = jnp.pad(ctq, ((0, 0), (0, NSB * TS - S1)), constant_values=1.0)
    ctq = ctq[..., None].astype(jnp.float32)
    return _build()(tok, ctq, seq_scores)
